# baseline (speedup 1.0000x reference)
import functools

import jax
import jax.numpy as jnp
from jax import lax
from jax.experimental import pallas as pl
from jax.experimental.pallas import tpu as pltpu
from jax.experimental.pallas import tpu_sc as plsc

EMB = 128
MAX_DEPTH = 20
N = 100000
C = 80
K = N // C
NC = 2
NS = 16
NW = NC * NS
LANES = 16


def _encoder(x0_hbm, x1_hbm, dep_hbm, ttab_hbm, atab_hbm, dtab_hbm, out_hbm,
             i0_v, i1_v, id_v, t_rows, a_rows, d_rows, sem):
    cid_core = lax.axis_index("c")
    sid = lax.axis_index("s")
    wid = sid * NC + cid_core

    def chunk_body(i, carry):
        cid = wid + i * NW
        base = cid * C
        pltpu.sync_copy(x0_hbm.at[pl.ds(base, C)], i0_v)
        pltpu.sync_copy(x1_hbm.at[pl.ds(base, C)], i1_v)
        pltpu.sync_copy(dep_hbm.at[pl.ds(base, C)], id_v)
        for j in range(C // LANES):
            sl = pl.ds(j * LANES, LANES)
            id_v[sl] = jnp.minimum(id_v[sl], MAX_DEPTH)
        cp1 = pltpu.async_copy(ttab_hbm.at[i0_v], t_rows, sem)
        cp2 = pltpu.async_copy(atab_hbm.at[i1_v], a_rows, sem)
        cp3 = pltpu.async_copy(dtab_hbm.at[id_v], d_rows, sem)
        cp1.wait()
        cp2.wait()
        cp3.wait()

        def add_row(r, c2):
            for k in range(EMB // LANES):
                sl = pl.ds(k * LANES, LANES)
                t_rows[r, sl] = t_rows[r, sl] + a_rows[r, sl] + d_rows[r, sl]
            return c2

        lax.fori_loop(0, C, add_row, 0)
        pltpu.sync_copy(t_rows, out_hbm.at[pl.ds(base, C)])
        return carry

    nchunks = (K // NW) + jnp.where(wid < (K % NW), 1, 0)
    lax.fori_loop(0, nchunks, chunk_body, 0)


@jax.jit
def _run(x0, x1, depth, type_table, attr_table, depth_table):
    enc = functools.partial(
        pl.kernel,
        mesh=plsc.VectorSubcoreMesh(core_axis_name="c", subcore_axis_name="s"),
        out_type=jax.ShapeDtypeStruct((N, EMB), jnp.float32),
        scratch_types=[
            pltpu.VMEM((C,), jnp.int32),
            pltpu.VMEM((C,), jnp.int32),
            pltpu.VMEM((C,), jnp.int32),
            pltpu.VMEM((C, EMB), jnp.float32),
            pltpu.VMEM((C, EMB), jnp.float32),
            pltpu.VMEM((C, EMB), jnp.float32),
            pltpu.SemaphoreType.DMA,
        ],
    )(_encoder)
    return enc(x0, x1, depth, type_table, attr_table, depth_table)


def kernel(x, depth, type_table, attr_table, depth_table):
    x0 = x[:, 0]
    x1 = x[:, 1]
    return _run(x0, x1, depth, type_table, attr_table, depth_table)


# speedup vs baseline: 7.1497x; 7.1497x over previous
import functools

import jax
import jax.numpy as jnp
from jax import lax
from jax.experimental import pallas as pl
from jax.experimental.pallas import tpu as pltpu
from jax.experimental.pallas import tpu_sc as plsc

EMB = 128
MAX_DEPTH = 20
NTYPE = 98
NATTR = 10030
N = 100000
C = 80
K = N // C
NC = 2
NS = 16
NW = NC * NS
LANES = 16
CPW = K // NW
MAXCH = CPW + 1


def _td_fuse_body(t_ref, d_ref, o_ref):
    o_ref[...] = t_ref[...][:, None, :] + d_ref[...][None, :, :]


def _encoder(td_hbm, atab_hbm, x0_hbm, x1_hbm, dep_hbm, out_hbm,
             x0_v, x1_v, dep_v, comb_v,
             td0, td1, a0, a1, s0, s1,
             isem, g0, g1, o0, o1):
    cc = lax.axis_index("c")
    ss = lax.axis_index("s")
    w = ss * NC + cc
    start = CPW * w + jnp.minimum(w, K % NW)
    n = CPW + jnp.where(w < K % NW, 1, 0)

    rbase = pl.multiple_of(start * C, 8)
    pltpu.async_copy(x0_hbm.at[pl.ds(rbase, CPW * C)], x0_v.at[pl.ds(0, CPW * C)], isem)
    pltpu.async_copy(x1_hbm.at[pl.ds(rbase, CPW * C)], x1_v.at[pl.ds(0, CPW * C)], isem)
    pltpu.async_copy(dep_hbm.at[pl.ds(rbase, CPW * C)], dep_v.at[pl.ds(0, CPW * C)], isem)
    pltpu.make_async_copy(x0_hbm.at[pl.ds(0, CPW * C)], x0_v.at[pl.ds(0, CPW * C)], isem).wait()
    pltpu.make_async_copy(x1_hbm.at[pl.ds(0, CPW * C)], x1_v.at[pl.ds(0, CPW * C)], isem).wait()
    pltpu.make_async_copy(dep_hbm.at[pl.ds(0, CPW * C)], dep_v.at[pl.ds(0, CPW * C)], isem).wait()

    @pl.when(w < K % NW)
    def _():
        ebase = pl.multiple_of((start + CPW) * C, 8)
        pltpu.sync_copy(x0_hbm.at[pl.ds(ebase, C)], x0_v.at[pl.ds(CPW * C, C)])
        pltpu.sync_copy(x1_hbm.at[pl.ds(ebase, C)], x1_v.at[pl.ds(CPW * C, C)])
        pltpu.sync_copy(dep_hbm.at[pl.ds(ebase, C)], dep_v.at[pl.ds(CPW * C, C)])

    def comb_vec(i, c2):
        sl = pl.ds(i * LANES, LANES)
        t = jnp.clip(x0_v[sl], 0, NTYPE - 1)
        d = jnp.clip(dep_v[sl], 0, MAX_DEPTH)
        comb_v[sl] = t * (MAX_DEPTH + 1) + d
        x1_v[sl] = jnp.clip(x1_v[sl], 0, NATTR - 1)
        return c2

    lax.fori_loop(0, MAXCH * C // LANES, comb_vec, 0)

    slots = ((td0, a0, s0, g0, o0), (td1, a1, s1, g1, o1))

    def fire(jj, tdb, ab, gsem):
        pltpu.async_copy(td_hbm.at[comb_v.at[pl.ds(jj * C, C)]], tdb, gsem)
        pltpu.async_copy(atab_hbm.at[x1_v.at[pl.ds(jj * C, C)]], ab, gsem)

    fire(0, td0, a0, g0)
    fire(1, td1, a1, g1)

    def pair(jp, carry):
        for b in range(2):
            tdb, ab, sb, gsem, osem = slots[b]
            jj = 2 * jp + b

            @pl.when(jj < n)
            def _():
                pltpu.make_async_copy(td_hbm.at[comb_v.at[pl.ds(jj * C, C)]], tdb, gsem).wait()
                pltpu.make_async_copy(atab_hbm.at[x1_v.at[pl.ds(jj * C, C)]], ab, gsem).wait()

                @pl.when(jp > 0)
                def _():
                    pltpu.make_async_copy(sb, out_hbm.at[pl.ds(0, C)], osem).wait()

                def add_row(r, c2):
                    for k in range(EMB // LANES):
                        sl = pl.ds(k * LANES, LANES)
                        sb[r, sl] = tdb[r, sl] + ab[r, sl]
                    return c2

                lax.fori_loop(0, C, add_row, 0)
                base = pl.multiple_of((start + jj) * C, 8)
                pltpu.async_copy(sb, out_hbm.at[pl.ds(base, C)], osem)

                @pl.when(jj + 2 < n)
                def _():
                    fire(jj + 2, tdb, ab, gsem)

        return carry

    lax.fori_loop(0, MAXCH // 2, pair, 0)
    pltpu.make_async_copy(s0, out_hbm.at[pl.ds(0, C)], o0).wait()
    pltpu.make_async_copy(s1, out_hbm.at[pl.ds(0, C)], o1).wait()


@jax.jit
def _run(x0, x1, depth, type_table, attr_table, depth_table):
    td3 = pl.pallas_call(
        _td_fuse_body,
        out_shape=jax.ShapeDtypeStruct((NTYPE, MAX_DEPTH + 1, EMB), jnp.float32),
    )(type_table, depth_table)
    td = td3.reshape(NTYPE * (MAX_DEPTH + 1), EMB)

    enc = functools.partial(
        pl.kernel,
        mesh=plsc.VectorSubcoreMesh(core_axis_name="c", subcore_axis_name="s"),
        out_type=jax.ShapeDtypeStruct((N, EMB), jnp.float32),
        scratch_types=[
            pltpu.VMEM((MAXCH * C,), jnp.int32),
            pltpu.VMEM((MAXCH * C,), jnp.int32),
            pltpu.VMEM((MAXCH * C,), jnp.int32),
            pltpu.VMEM((MAXCH * C,), jnp.int32),
            pltpu.VMEM((C, EMB), jnp.float32),
            pltpu.VMEM((C, EMB), jnp.float32),
            pltpu.VMEM((C, EMB), jnp.float32),
            pltpu.VMEM((C, EMB), jnp.float32),
            pltpu.VMEM((C, EMB), jnp.float32),
            pltpu.VMEM((C, EMB), jnp.float32),
            pltpu.SemaphoreType.DMA,
            pltpu.SemaphoreType.DMA,
            pltpu.SemaphoreType.DMA,
            pltpu.SemaphoreType.DMA,
            pltpu.SemaphoreType.DMA,
        ],
    )(_encoder)
    return enc(td, attr_table, x0, x1, depth)


def kernel(x, depth, type_table, attr_table, depth_table):
    return _run(x[:, 0], x[:, 1], depth, type_table, attr_table, depth_table)
